# Initial kernel scaffold; baseline (speedup 1.0000x reference)
import functools

import jax
import jax.numpy as jnp
from jax import lax
from jax.experimental import pallas as pl
from jax.experimental.pallas import tpu as pltpu
from jax.experimental.pallas import tpu_sc as plsc

NC = 2
NS = 16
NW = NC * NS

CHUNK = 128
CHUNKS_PER_GROUP = 20
GROUP = CHUNK * CHUNKS_PER_GROUP


def _gather_kernel(B, D, table_hbm, idx_hbm, out_hbm, idx_v, rows_v, sem):
    b_per_w = B // NW
    n_groups = b_per_w // GROUP
    wid = lax.axis_index("s") * NC + lax.axis_index("c")
    base = wid * b_per_w
    pltpu.sync_copy(idx_hbm.at[pl.ds(base, b_per_w)], idx_v)

    @pl.loop(0, n_groups)
    def _(g):
        g0 = g * GROUP
        copies = [
            pltpu.async_copy(
                table_hbm.at[idx_v.at[pl.ds(g0 + j * CHUNK, CHUNK)]],
                rows_v.at[pl.ds(j * CHUNK, CHUNK), :],
                sem,
            )
            for j in range(CHUNKS_PER_GROUP)
        ]
        for c in copies:
            c.wait()
        pltpu.sync_copy(rows_v, out_hbm.at[pl.ds(base + g0, GROUP)])


def kernel(phonemes, table):
    S0, S1 = phonemes.shape
    B = S0 * S1
    V, D = table.shape
    idx = phonemes.reshape(B).astype(jnp.int32)

    mesh = plsc.VectorSubcoreMesh(core_axis_name="c", subcore_axis_name="s")
    b_per_w = B // NW

    k = pl.kernel(
        functools.partial(_gather_kernel, B, D),
        out_type=jax.ShapeDtypeStruct((B, D), table.dtype),
        mesh=mesh,
        scratch_types=[
            pltpu.VMEM((b_per_w,), jnp.int32),
            pltpu.VMEM((GROUP, D), jnp.float32),
            pltpu.SemaphoreType.DMA,
        ],
    )
    out = k(table, idx)
    return out.reshape(S0, S1, D)


# speedup vs baseline: 5.6157x; 5.6157x over previous
import functools

import jax
import jax.numpy as jnp
from jax import lax
from jax.experimental import pallas as pl
from jax.experimental.pallas import tpu as pltpu
from jax.experimental.pallas import tpu_sc as plsc

NC = 2
NS = 16
NW = NC * NS

CHUNK = 128
CHUNKS_PER_GROUP = 5
GROUP = CHUNK * CHUNKS_PER_GROUP
LINE = 128


def _gather_kernel(B, D, table_hbm, idx_hbm, out_hbm, idx_v, rows_v, sem):
    b_per_w = B // NW
    n_groups = b_per_w // GROUP
    wid = lax.axis_index("s") * NC + lax.axis_index("c")
    base = wid * b_per_w
    pltpu.sync_copy(idx_hbm.at[pl.ds(base, b_per_w)], idx_v)

    @pl.loop(0, n_groups)
    def _(g):
        g0 = g * GROUP
        copies = [
            pltpu.async_copy(
                table_hbm.at[idx_v.at[pl.ds(g0 + j * CHUNK, CHUNK)]],
                rows_v.at[pl.ds(j * CHUNK, CHUNK), :],
                sem,
            )
            for j in range(CHUNKS_PER_GROUP)
        ]
        for c in copies:
            c.wait()
        pltpu.sync_copy(rows_v, out_hbm.at[pl.ds(base + g0, GROUP)])


def kernel(phonemes, table):
    S0, S1 = phonemes.shape
    B = S0 * S1
    V, D = table.shape
    idx = phonemes.reshape(B).astype(jnp.int32)
    table_pad = jnp.pad(table, ((0, 0), (0, LINE - D)))

    mesh = plsc.VectorSubcoreMesh(core_axis_name="c", subcore_axis_name="s")
    b_per_w = B // NW

    k = pl.kernel(
        functools.partial(_gather_kernel, B, D),
        out_type=jax.ShapeDtypeStruct((B, LINE), table.dtype),
        mesh=mesh,
        scratch_types=[
            pltpu.VMEM((b_per_w,), jnp.int32),
            pltpu.VMEM((GROUP, LINE), jnp.float32),
            pltpu.SemaphoreType.DMA,
        ],
    )
    out = k(table_pad, idx)
    return out[:, :D].reshape(S0, S1, D)
